# baseline (speedup 1.0000x reference)
import functools

import jax
import jax.numpy as jnp
import numpy as np
from jax import lax
from jax.experimental import pallas as pl
from jax.experimental.pallas import tpu as pltpu
from jax.experimental.pallas import tpu_sc as plsc

_EPS = 1e-5
_NC = 2
_NS = 16
_NW = _NC * _NS
_IT = False


def _pick_chunk(n, cap=128):
    for c in range(cap, 7, -8):
        if n % c == 0:
            return c
    raise ValueError(f"no 8-aligned chunk divides {n}")


def _bn_cols(z, g, bt):
    m = jnp.mean(z, axis=0, keepdims=True)
    var = jnp.mean(jnp.square(z - m), axis=0, keepdims=True)
    return g * (z - m) * lax.rsqrt(var + _EPS) + bt


def _emb_body(x_ref, w_ref, b_ref, g_ref, bt_ref, o_ref):
    z = jnp.dot(x_ref[...], w_ref[...], preferred_element_type=jnp.float32)
    zn = _bn_cols(z + b_ref[...], g_ref[...], bt_ref[...])
    o_ref[...] = zn * jax.nn.sigmoid(zn)


def _tables_body(v_ref, ws_ref, wd_ref, bd_ref, ts_ref, td_ref):
    v = v_ref[...]
    ts_ref[...] = jnp.dot(v, ws_ref[...], preferred_element_type=jnp.float32)
    td_ref[...] = jnp.dot(v, wd_ref[...],
                          preferred_element_type=jnp.float32) + bd_ref[...]


def _stats_body(ss_ref, ef_ref, we_ref, o_ref, acc_ref):
    i = pl.program_id(0)

    @pl.when(i == 0)
    def _():
        acc_ref[...] = jnp.zeros_like(acc_ref)

    z = ss_ref[...].astype(jnp.float32) + jnp.dot(
        ef_ref[...], we_ref[...], preferred_element_type=jnp.float32)
    acc_ref[0:1, :] += jnp.sum(z, axis=0, keepdims=True)
    acc_ref[1:2, :] += jnp.sum(z * z, axis=0, keepdims=True)

    @pl.when(i == pl.num_programs(0) - 1)
    def _():
        o_ref[...] = acc_ref[...]


def _apply_body(ss_ref, ef_ref, we_ref, st_ref, g_ref, bt_ref, o_ref, *,
                nedges, dm):
    z = ss_ref[...].astype(jnp.float32) + jnp.dot(
        ef_ref[...], we_ref[...], preferred_element_type=jnp.float32)
    mean = st_ref[0:1, :] / nedges
    var = st_ref[1:2, :] / nedges - mean * mean
    alpha = g_ref[...] * lax.rsqrt(var + _EPS)
    beta = bt_ref[...] - mean * alpha
    zn = z * alpha + beta
    o_ref[...] = jax.nn.sigmoid(zn[:, :dm]) * jax.nn.softplus(zn[:, dm:])


def _update_body(ag_ref, v_ref, g_ref, bt_ref, o_ref):
    a = ag_ref[0] + ag_ref[1]
    an = _bn_cols(a, g_ref[...], bt_ref[...])
    o_ref[...] = jax.nn.softplus(an + v_ref[...])


def _tail_body(v_ref, gid_ref, w1_ref, b1_ref, g1_ref, bt1_ref,
               w2_ref, b2_ref, g2_ref, bt2_ref, wp_ref, bp_ref, o_ref, *, g):
    ids = gid_ref[...]
    iota = lax.broadcasted_iota(jnp.int32, (g, ids.shape[1]), 0)
    onehot = jnp.where(ids == iota, 1.0, 0.0).astype(jnp.float32)
    sums = jnp.dot(onehot, v_ref[...], preferred_element_type=jnp.float32,
                   precision=lax.Precision.HIGHEST)
    cnt = jnp.sum(onehot, axis=1, keepdims=True)
    x = sums / jnp.maximum(cnt, 1.0)
    for w, b, gg, bt in ((w1_ref, b1_ref, g1_ref, bt1_ref),
                         (w2_ref, b2_ref, g2_ref, bt2_ref)):
        z = jnp.dot(x, w[...], preferred_element_type=jnp.float32) + b[...]
        zn = _bn_cols(z, gg[...], bt[...])
        x = zn * jax.nn.sigmoid(zn)
    o_ref[...] = jnp.dot(x, wp_ref[...],
                         preferred_element_type=jnp.float32) + bp_ref[...]


def _make_gather(e_total, d2, ch):
    epw = e_total // _NW
    nch = epw // ch
    mesh = plsc.VectorSubcoreMesh(core_axis_name="c", subcore_axis_name="s",
                                  num_cores=_NC, num_subcores=_NS)

    @functools.partial(
        pl.kernel,
        out_type=jax.ShapeDtypeStruct((e_total, d2), jnp.float32),
        mesh=mesh,
        interpret=_IT,
        scratch_types=[
            pltpu.VMEM((nch, ch), jnp.int32),
            pltpu.VMEM((nch, ch), jnp.int32),
            [pltpu.VMEM((ch, d2), jnp.float32)] * 2,
            [pltpu.VMEM((ch, d2), jnp.float32)] * 2,
            [pltpu.VMEM((ch, d2), jnp.float32)] * 2,
            [pltpu.SemaphoreType.DMA] * 2,
            [pltpu.SemaphoreType.DMA] * 2,
            [pltpu.SemaphoreType.DMA] * 2,
        ],
    )
    def gather_kernel(src_hbm, dst_hbm, ts_hbm, td_hbm, ss_hbm,
                      sidx, didx, bufs, bufd, obuf, gsa, gsb, osem):
        wid = lax.axis_index("s") * _NC + lax.axis_index("c")
        base = wid * epw
        pltpu.sync_copy(src_hbm.at[wid], sidx)
        pltpu.sync_copy(dst_hbm.at[wid], didx)

        def fire(b, c):
            pltpu.async_copy(ts_hbm.at[sidx.at[c]], bufs[b], gsa[b])
            pltpu.async_copy(td_hbm.at[didx.at[c]], bufd[b], gsb[b])

        def wait_g(b, c):
            pltpu.make_async_copy(ts_hbm.at[sidx.at[c]], bufs[b],
                                  gsa[b]).wait()
            pltpu.make_async_copy(td_hbm.at[didx.at[c]], bufd[b],
                                  gsb[b]).wait()

        def wait_o(b):
            pltpu.make_async_copy(
                obuf[b], ss_hbm.at[pl.ds(base, ch)], osem[b]).wait()

        def add_and_write(b, c, wait_pred):
            wait_g(b, c)

            @pl.when(wait_pred)
            def _():
                wait_o(b)

            def add_row(j, c2):
                for k in range(d2 // 16):
                    sl = pl.ds(k * 16, 16)
                    obuf[b][j, sl] = bufs[b][j, sl] + bufd[b][j, sl]
                return c2

            lax.fori_loop(0, ch, add_row, 0)
            pltpu.async_copy(obuf[b], ss_hbm.at[pl.ds(base + c * ch, ch)],
                             osem[b])

        fire(0, 0)

        def do_pair(i2, carry):
            c0 = 2 * i2
            fire(1, c0 + 1)
            add_and_write(0, c0, i2 > 0)

            @pl.when(c0 + 2 < nch)
            def _():
                fire(0, c0 + 2)
            add_and_write(1, c0 + 1, i2 > 0)
            return carry

        lax.fori_loop(0, nch // 2, do_pair, 0)

        if nch % 2:
            add_and_write(0, nch - 1, nch > 1)
            wait_o(1)
            wait_o(0)
        else:
            wait_o(0)
            wait_o(1)

    return gather_kernel


def _make_scatter(e_total, n_nodes, d, ch):
    epw = e_total // _NW
    nch = epw // ch
    rc = _pick_chunk(n_nodes)
    nrc = n_nodes // rc
    kmax = (nrc + _NS - 1) // _NS
    mesh = plsc.VectorSubcoreMesh(core_axis_name="c", subcore_axis_name="s",
                                  num_cores=_NC, num_subcores=_NS)

    @functools.partial(
        pl.kernel,
        out_type=jax.ShapeDtypeStruct((_NC, n_nodes, d), jnp.float32),
        mesh=mesh,
        interpret=_IT,
        scratch_types=[
            [pltpu.VMEM((ch,), jnp.int32)] * 2,
            [pltpu.VMEM((ch, d), jnp.float32)] * 2,
            pltpu.VMEM((rc, d), jnp.float32),
            pltpu.VMEM_SHARED((n_nodes, d), jnp.float32),
            [pltpu.SemaphoreType.DMA] * 2,
            [pltpu.SemaphoreType.DMA] * 2,
        ],
    )
    def scatter_kernel(dst_hbm, h_hbm, out_hbm, didx, hbuf, zbuf, agg_sh,
                       sem, isem):
        cid = lax.axis_index("c")
        sid = lax.axis_index("s")
        wid = sid * _NC + cid
        base = wid * epw

        def z_row(j, c):
            for k in range(d // 16):
                zbuf[j, pl.ds(k * 16, 16)] = jnp.zeros((16,), jnp.float32)
            return c

        lax.fori_loop(0, rc, z_row, 0)

        def z_chunk(kk, c):
            cidx = sid + kk * _NS

            @pl.when(cidx < nrc)
            def _():
                pltpu.sync_copy(zbuf, agg_sh.at[pl.ds(cidx * rc, rc)])
            return c

        lax.fori_loop(0, kmax, z_chunk, 0)
        plsc.subcore_barrier()

        def fire(b, c):
            pltpu.async_copy(dst_hbm.at[wid, c], didx[b], isem[b])
            pltpu.async_copy(h_hbm.at[pl.ds(base + c * ch, ch)],
                             hbuf[b], sem[b])

        def scat(b, c):
            pltpu.make_async_copy(
                dst_hbm.at[wid, 0], didx[b], isem[b]).wait()
            pltpu.make_async_copy(
                h_hbm.at[pl.ds(base, ch)], hbuf[b], sem[b]).wait()
            pltpu.sync_copy(hbuf[b], agg_sh.at[didx[b]], add=True)

        fire(0, 0)

        def do_pair(i2, c):
            c0 = 2 * i2
            fire(1, c0 + 1)
            scat(0, c0)

            @pl.when(c0 + 2 < nch)
            def _():
                fire(0, c0 + 2)
            scat(1, c0 + 1)
            return c

        lax.fori_loop(0, nch // 2, do_pair, 0)
        if nch % 2:
            scat(0, nch - 1)
        plsc.subcore_barrier()

        def w_chunk(kk, c):
            cidx = sid + kk * _NS

            @pl.when(cidx < nrc)
            def _():
                sl = pl.ds(cidx * rc, rc)
                pltpu.sync_copy(agg_sh.at[sl], out_hbm.at[cid].at[sl])
            return c

        lax.fori_loop(0, kmax, w_chunk, 0)

    return scatter_kernel


def _tc(body, out_shape, **kw):
    return pl.pallas_call(body, out_shape=out_shape, interpret=_IT, **kw)


def kernel(node_feats, edge_feats, edge_index, node_graph_ids, params):
    n, dn = node_feats.shape
    e, de = edge_feats.shape
    demb = params['emb']['W'].shape[1]
    d2 = 2 * demb
    g_graphs = 64
    f32 = jnp.float32

    def row(x):
        return x.reshape(1, -1)

    src = edge_index[0]
    dst = edge_index[1]

    pe = params['emb']
    v = _tc(_emb_body, jax.ShapeDtypeStruct((n, demb), f32))(
        node_feats, pe['W'], row(pe['b']), row(pe['g']), row(pe['bt']))

    epw = e // _NW
    ch = 40
    nch = epw // ch
    src3 = src.reshape(_NW, nch, ch)
    dst3 = dst.reshape(_NW, nch, ch)
    gather = _make_gather(e, d2, ch)
    scatter = _make_scatter(e, n, demb, ch)

    eb = _pick_chunk(e, cap=6400)
    grid = (e // eb,)
    ss_spec = pl.BlockSpec((eb, d2), lambda i: (i, 0))
    ef_spec = pl.BlockSpec((eb, de), lambda i: (i, 0))
    full = lambda shape: pl.BlockSpec(shape, lambda i: (0, 0))

    stats_call = _tc(
        _stats_body, jax.ShapeDtypeStruct((2, d2), f32), grid=grid,
        in_specs=[ss_spec, ef_spec, full((de, d2))],
        out_specs=full((2, d2)),
        scratch_shapes=[pltpu.VMEM((2, d2), f32)])

    apply_call = _tc(
        functools.partial(_apply_body, nedges=float(e), dm=demb),
        jax.ShapeDtypeStruct((e, demb), f32), grid=grid,
        in_specs=[ss_spec, ef_spec, full((de, d2)), full((2, d2)),
                  full((1, d2)), full((1, d2))],
        out_specs=pl.BlockSpec((eb, demb), lambda i: (i, 0)))

    tables_call = _tc(
        _tables_body, [jax.ShapeDtypeStruct((n, d2), f32)] * 2)

    update_call = _tc(_update_body, jax.ShapeDtypeStruct((n, demb), f32))

    for cp in params['convs']:
        w1 = cp['mlp']['W']
        w2 = cp['screen']['W']
        ws = jnp.concatenate([w1[:demb], w2[:demb]], axis=1)
        wd = jnp.concatenate([w1[demb:d2], w2[demb:d2]], axis=1)
        we = jnp.concatenate([w1[d2:], w2[d2:]], axis=1)
        bd = row(jnp.concatenate([cp['mlp']['b'], cp['screen']['b']]))
        gc = row(jnp.concatenate([cp['mlp']['g'], cp['screen']['g']]))
        btc = row(jnp.concatenate([cp['mlp']['bt'], cp['screen']['bt']]))

        ts, td = tables_call(v, ws, wd, bd)
        ss = gather(src3, dst3, ts, td)
        st = stats_call(ss, edge_feats, we)
        h = apply_call(ss, edge_feats, we, st, gc, btc)
        aggp = scatter(dst3, h)
        v = update_call(aggp, v, row(cp['bn_g']), row(cp['bn_bt']))

    fp1, fp2 = params['fcs']
    pp = params['pred']
    out = _tc(functools.partial(_tail_body, g=g_graphs),
              jax.ShapeDtypeStruct((g_graphs, 1), f32))(
        v, node_graph_ids.reshape(1, -1).astype(jnp.int32),
        fp1['W'], row(fp1['b']), row(fp1['g']), row(fp1['bt']),
        fp2['W'], row(fp2['b']), row(fp2['g']), row(fp2['bt']),
        pp['W'], row(pp['b']))
    return out


# speedup vs baseline: 1.0268x; 1.0268x over previous
import functools

import jax
import jax.numpy as jnp
import numpy as np
from jax import lax
from jax.experimental import pallas as pl
from jax.experimental.pallas import tpu as pltpu
from jax.experimental.pallas import tpu_sc as plsc

_EPS = 1e-5
_NC = 2
_NS = 16
_NW = _NC * _NS
_IT = False


def _pick_chunk(n, cap=128):
    for c in range(cap, 7, -8):
        if n % c == 0:
            return c
    raise ValueError(f"no 8-aligned chunk divides {n}")


def _bn_cols(z, g, bt):
    m = jnp.mean(z, axis=0, keepdims=True)
    var = jnp.mean(jnp.square(z - m), axis=0, keepdims=True)
    return g * (z - m) * lax.rsqrt(var + _EPS) + bt


def _emb_body(x_ref, w_ref, b_ref, g_ref, bt_ref, o_ref):
    z = jnp.dot(x_ref[...], w_ref[...], preferred_element_type=jnp.float32)
    zn = _bn_cols(z + b_ref[...], g_ref[...], bt_ref[...])
    o_ref[...] = zn * jax.nn.sigmoid(zn)


def _tables_body(v_ref, ws_ref, wd_ref, bd_ref, ts_ref, td_ref):
    v = v_ref[...]
    ts_ref[...] = jnp.dot(v, ws_ref[...], preferred_element_type=jnp.float32)
    td_ref[...] = jnp.dot(v, wd_ref[...],
                          preferred_element_type=jnp.float32) + bd_ref[...]


def _stats_body(ss_ref, ef_ref, we_ref, o_ref, acc_ref):
    i = pl.program_id(0)

    @pl.when(i == 0)
    def _():
        acc_ref[...] = jnp.zeros_like(acc_ref)

    z = ss_ref[...].astype(jnp.float32) + jnp.dot(
        ef_ref[...], we_ref[...], preferred_element_type=jnp.float32)
    acc_ref[0:1, :] += jnp.sum(z, axis=0, keepdims=True)
    acc_ref[1:2, :] += jnp.sum(z * z, axis=0, keepdims=True)

    @pl.when(i == pl.num_programs(0) - 1)
    def _():
        o_ref[...] = acc_ref[...]


def _apply_body(ss_ref, ef_ref, we_ref, st0_ref, st1_ref, g_ref, bt_ref,
                o_ref, *, nedges, dm):
    z = ss_ref[...].astype(jnp.float32) + jnp.dot(
        ef_ref[...], we_ref[...], preferred_element_type=jnp.float32)
    st = st0_ref[...] + st1_ref[...]
    mean = st[0:1, :] / nedges
    var = st[1:2, :] / nedges - mean * mean
    alpha = g_ref[...] * lax.rsqrt(var + _EPS)
    beta = bt_ref[...] - mean * alpha
    zn = z * alpha + beta
    o_ref[...] = jax.nn.sigmoid(zn[:, :dm]) * jax.nn.softplus(zn[:, dm:])


def _update_body(ag0_ref, ag1_ref, v_ref, g_ref, bt_ref, o_ref):
    a = (ag0_ref[0] + ag0_ref[1]) + (ag1_ref[0] + ag1_ref[1])
    an = _bn_cols(a, g_ref[...], bt_ref[...])
    o_ref[...] = jax.nn.softplus(an + v_ref[...])


def _tail_body(v_ref, gid_ref, w1_ref, b1_ref, g1_ref, bt1_ref,
               w2_ref, b2_ref, g2_ref, bt2_ref, wp_ref, bp_ref, o_ref, *, g):
    ids = gid_ref[...]
    iota = lax.broadcasted_iota(jnp.int32, (g, ids.shape[1]), 0)
    onehot = jnp.where(ids == iota, 1.0, 0.0).astype(jnp.float32)
    sums = jnp.dot(onehot, v_ref[...], preferred_element_type=jnp.float32,
                   precision=lax.Precision.HIGHEST)
    cnt = jnp.sum(onehot, axis=1, keepdims=True)
    x = sums / jnp.maximum(cnt, 1.0)
    for w, b, gg, bt in ((w1_ref, b1_ref, g1_ref, bt1_ref),
                         (w2_ref, b2_ref, g2_ref, bt2_ref)):
        z = jnp.dot(x, w[...], preferred_element_type=jnp.float32) + b[...]
        zn = _bn_cols(z, gg[...], bt[...])
        x = zn * jax.nn.sigmoid(zn)
    o_ref[...] = jnp.dot(x, wp_ref[...],
                         preferred_element_type=jnp.float32) + bp_ref[...]


def _make_gather(e_total, d2, ch):
    epw = e_total // _NW
    nch = epw // ch
    mesh = plsc.VectorSubcoreMesh(core_axis_name="c", subcore_axis_name="s",
                                  num_cores=_NC, num_subcores=_NS)

    @functools.partial(
        pl.kernel,
        out_type=jax.ShapeDtypeStruct((e_total, d2), jnp.float32),
        mesh=mesh,
        interpret=_IT,
        scratch_types=[
            pltpu.VMEM((nch, ch), jnp.int32),
            pltpu.VMEM((nch, ch), jnp.int32),
            [pltpu.VMEM((ch, d2), jnp.float32)] * 2,
            [pltpu.VMEM((ch, d2), jnp.float32)] * 2,
            [pltpu.VMEM((ch, d2), jnp.float32)] * 2,
            [pltpu.SemaphoreType.DMA] * 2,
            [pltpu.SemaphoreType.DMA] * 2,
            [pltpu.SemaphoreType.DMA] * 2,
        ],
    )
    def gather_kernel(src_hbm, dst_hbm, ts_hbm, td_hbm, ss_hbm,
                      sidx, didx, bufs, bufd, obuf, gsa, gsb, osem):
        wid = lax.axis_index("s") * _NC + lax.axis_index("c")
        base = wid * epw
        pltpu.sync_copy(src_hbm.at[wid], sidx)
        pltpu.sync_copy(dst_hbm.at[wid], didx)

        def fire(b, c):
            pltpu.async_copy(ts_hbm.at[sidx.at[c]], bufs[b], gsa[b])
            pltpu.async_copy(td_hbm.at[didx.at[c]], bufd[b], gsb[b])

        def wait_g(b, c):
            pltpu.make_async_copy(ts_hbm.at[sidx.at[c]], bufs[b],
                                  gsa[b]).wait()
            pltpu.make_async_copy(td_hbm.at[didx.at[c]], bufd[b],
                                  gsb[b]).wait()

        def wait_o(b):
            pltpu.make_async_copy(
                obuf[b], ss_hbm.at[pl.ds(base, ch)], osem[b]).wait()

        def add_and_write(b, c, wait_pred):
            wait_g(b, c)

            @pl.when(wait_pred)
            def _():
                wait_o(b)

            def add_row(j, c2):
                for k in range(d2 // 16):
                    sl = pl.ds(k * 16, 16)
                    obuf[b][j, sl] = bufs[b][j, sl] + bufd[b][j, sl]
                return c2

            lax.fori_loop(0, ch, add_row, 0)
            pltpu.async_copy(obuf[b], ss_hbm.at[pl.ds(base + c * ch, ch)],
                             osem[b])

        fire(0, 0)

        def do_pair(i2, carry):
            c0 = 2 * i2
            fire(1, c0 + 1)
            add_and_write(0, c0, i2 > 0)

            @pl.when(c0 + 2 < nch)
            def _():
                fire(0, c0 + 2)
            add_and_write(1, c0 + 1, i2 > 0)
            return carry

        lax.fori_loop(0, nch // 2, do_pair, 0)

        if nch % 2:
            add_and_write(0, nch - 1, nch > 1)
            wait_o(1)
            wait_o(0)
        else:
            wait_o(0)
            wait_o(1)

    return gather_kernel


def _make_scatter(e_total, n_nodes, d, ch):
    epw = e_total // _NW
    nch = epw // ch
    rc = _pick_chunk(n_nodes)
    nrc = n_nodes // rc
    kmax = (nrc + _NS - 1) // _NS
    mesh = plsc.VectorSubcoreMesh(core_axis_name="c", subcore_axis_name="s",
                                  num_cores=_NC, num_subcores=_NS)

    @functools.partial(
        pl.kernel,
        out_type=jax.ShapeDtypeStruct((_NC, n_nodes, d), jnp.float32),
        mesh=mesh,
        interpret=_IT,
        scratch_types=[
            [pltpu.VMEM((ch,), jnp.int32)] * 2,
            [pltpu.VMEM((ch, d), jnp.float32)] * 2,
            pltpu.VMEM((rc, d), jnp.float32),
            pltpu.VMEM_SHARED((n_nodes, d), jnp.float32),
            [pltpu.SemaphoreType.DMA] * 2,
            [pltpu.SemaphoreType.DMA] * 2,
        ],
    )
    def scatter_kernel(dst_hbm, h_hbm, out_hbm, didx, hbuf, zbuf, agg_sh,
                       sem, isem):
        cid = lax.axis_index("c")
        sid = lax.axis_index("s")
        wid = sid * _NC + cid
        base = wid * epw

        def z_row(j, c):
            for k in range(d // 16):
                zbuf[j, pl.ds(k * 16, 16)] = jnp.zeros((16,), jnp.float32)
            return c

        lax.fori_loop(0, rc, z_row, 0)

        def z_chunk(kk, c):
            cidx = sid + kk * _NS

            @pl.when(cidx < nrc)
            def _():
                pltpu.sync_copy(zbuf, agg_sh.at[pl.ds(cidx * rc, rc)])
            return c

        lax.fori_loop(0, kmax, z_chunk, 0)
        plsc.subcore_barrier()

        def fire(b, c):
            pltpu.async_copy(dst_hbm.at[wid, c], didx[b], isem[b])
            pltpu.async_copy(h_hbm.at[pl.ds(base + c * ch, ch)],
                             hbuf[b], sem[b])

        def scat(b, c):
            pltpu.make_async_copy(
                dst_hbm.at[wid, 0], didx[b], isem[b]).wait()
            pltpu.make_async_copy(
                h_hbm.at[pl.ds(base, ch)], hbuf[b], sem[b]).wait()
            pltpu.sync_copy(hbuf[b], agg_sh.at[didx[b]], add=True)

        fire(0, 0)

        def do_pair(i2, c):
            c0 = 2 * i2
            fire(1, c0 + 1)
            scat(0, c0)

            @pl.when(c0 + 2 < nch)
            def _():
                fire(0, c0 + 2)
            scat(1, c0 + 1)
            return c

        lax.fori_loop(0, nch // 2, do_pair, 0)
        if nch % 2:
            scat(0, nch - 1)
        plsc.subcore_barrier()

        def w_chunk(kk, c):
            cidx = sid + kk * _NS

            @pl.when(cidx < nrc)
            def _():
                sl = pl.ds(cidx * rc, rc)
                pltpu.sync_copy(agg_sh.at[sl], out_hbm.at[cid].at[sl])
            return c

        lax.fori_loop(0, kmax, w_chunk, 0)

    return scatter_kernel


def _tc(body, out_shape, **kw):
    return pl.pallas_call(body, out_shape=out_shape, interpret=_IT, **kw)


def kernel(node_feats, edge_feats, edge_index, node_graph_ids, params):
    n, dn = node_feats.shape
    e, de = edge_feats.shape
    demb = params['emb']['W'].shape[1]
    d2 = 2 * demb
    g_graphs = 64
    f32 = jnp.float32

    def row(x):
        return x.reshape(1, -1)

    src = edge_index[0]
    dst = edge_index[1]

    pe = params['emb']
    v = _tc(_emb_body, jax.ShapeDtypeStruct((n, demb), f32))(
        node_feats, pe['W'], row(pe['b']), row(pe['g']), row(pe['bt']))

    eh = e // 2
    epw = eh // _NW
    ch = 40
    nch = epw // ch
    halves = []
    for hx in range(2):
        sl = slice(hx * eh, (hx + 1) * eh)
        halves.append((src[sl].reshape(_NW, nch, ch),
                       dst[sl].reshape(_NW, nch, ch),
                       edge_feats[sl]))
    gather = _make_gather(eh, d2, ch)
    scatter = _make_scatter(eh, n, demb, ch)

    eb = _pick_chunk(eh, cap=6400)
    grid = (eh // eb,)
    ss_spec = pl.BlockSpec((eb, d2), lambda i: (i, 0))
    ef_spec = pl.BlockSpec((eb, de), lambda i: (i, 0))
    full = lambda shape: pl.BlockSpec(shape, lambda i: (0, 0))

    stats_call = _tc(
        _stats_body, jax.ShapeDtypeStruct((2, d2), f32), grid=grid,
        in_specs=[ss_spec, ef_spec, full((de, d2))],
        out_specs=full((2, d2)),
        scratch_shapes=[pltpu.VMEM((2, d2), f32)])

    apply_call = _tc(
        functools.partial(_apply_body, nedges=float(e), dm=demb),
        jax.ShapeDtypeStruct((eh, demb), f32), grid=grid,
        in_specs=[ss_spec, ef_spec, full((de, d2)), full((2, d2)),
                  full((2, d2)), full((1, d2)), full((1, d2))],
        out_specs=pl.BlockSpec((eb, demb), lambda i: (i, 0)))

    tables_call = _tc(
        _tables_body, [jax.ShapeDtypeStruct((n, d2), f32)] * 2)

    update_call = _tc(_update_body, jax.ShapeDtypeStruct((n, demb), f32))

    for cp in params['convs']:
        w1 = cp['mlp']['W']
        w2 = cp['screen']['W']
        ws = jnp.concatenate([w1[:demb], w2[:demb]], axis=1)
        wd = jnp.concatenate([w1[demb:d2], w2[demb:d2]], axis=1)
        we = jnp.concatenate([w1[d2:], w2[d2:]], axis=1)
        bd = row(jnp.concatenate([cp['mlp']['b'], cp['screen']['b']]))
        gc = row(jnp.concatenate([cp['mlp']['g'], cp['screen']['g']]))
        btc = row(jnp.concatenate([cp['mlp']['bt'], cp['screen']['bt']]))

        ts, td = tables_call(v, ws, wd, bd)
        ss0 = gather(halves[0][0], halves[0][1], ts, td)
        ss1 = gather(halves[1][0], halves[1][1], ts, td)
        st0 = stats_call(ss0, halves[0][2], we)
        st1 = stats_call(ss1, halves[1][2], we)
        h0 = apply_call(ss0, halves[0][2], we, st0, st1, gc, btc)
        agg0 = scatter(halves[0][1], h0)
        h1 = apply_call(ss1, halves[1][2], we, st0, st1, gc, btc)
        agg1 = scatter(halves[1][1], h1)
        v = update_call(agg0, agg1, v, row(cp['bn_g']), row(cp['bn_bt']))

    fp1, fp2 = params['fcs']
    pp = params['pred']
    out = _tc(functools.partial(_tail_body, g=g_graphs),
              jax.ShapeDtypeStruct((g_graphs, 1), f32))(
        v, node_graph_ids.reshape(1, -1).astype(jnp.int32),
        fp1['W'], row(fp1['b']), row(fp1['g']), row(fp1['bt']),
        fp2['W'], row(fp2['b']), row(fp2['g']), row(fp2['bt']),
        pp['W'], row(pp['b']))
    return out
